# baseline (speedup 1.0000x reference)
import functools

import jax
import jax.numpy as jnp
from jax import lax
from jax.experimental import pallas as pl
from jax.experimental.pallas import tpu as pltpu
from jax.experimental.pallas import tpu_sc as plsc

E = 320000
N = 10000
R = 16
C = 128


_BE = 8000


def _proj_body(ea_ref, w_ref, b_ref, o_ref):
    o_ref[...] = (
        jnp.dot(ea_ref[...], w_ref[...], preferred_element_type=jnp.float32)
        + b_ref[...]
    )


def _proj_tc(edge_attr, W, b):
    return pl.pallas_call(
        _proj_body,
        grid=(E // _BE,),
        in_specs=[
            pl.BlockSpec((_BE, R), lambda i: (i, 0)),
            pl.BlockSpec((R, C), lambda i: (0, 0)),
            pl.BlockSpec((1, C), lambda i: (0, 0)),
        ],
        out_specs=pl.BlockSpec((_BE, C), lambda i: (i, 0)),
        out_shape=jax.ShapeDtypeStruct((E, C), jnp.float32),
    )(edge_attr, W, b.reshape(1, C))


_K = 40
_NBUF = 2


def _combine_sc(senders, receivers, proj, x):
    info = plsc.get_sparse_core_info()
    nc = info.num_cores
    nw = nc * info.num_subcores
    per_w = E // nw
    n_chunk = per_w // _K

    mesh = plsc.VectorSubcoreMesh(core_axis_name="c", subcore_axis_name="s")

    ns = info.num_subcores
    rpt = (N // ns) // 8 * 8
    rpt_last = N - (ns - 1) * rpt

    scratch = [pltpu.VMEM_SHARED((N, C), jnp.float32)]
    for _ in range(_NBUF):
        scratch += [
            pltpu.VMEM((_K,), jnp.int32),
            pltpu.VMEM((_K,), jnp.int32),
            pltpu.VMEM((_K, C), jnp.float32),
            pltpu.VMEM((_K, C), jnp.float32),
            pltpu.VMEM((_K, C), jnp.float32),
            pltpu.VMEM((_K, C), jnp.float32),
            pltpu.SemaphoreType.DMA,
            pltpu.SemaphoreType.DMA,
            pltpu.SemaphoreType.DMA,
            pltpu.SemaphoreType.DMA,
        ]

    @functools.partial(
        pl.kernel,
        mesh=mesh,
        out_type=jax.ShapeDtypeStruct((E, C), jnp.float32),
        scratch_types=scratch,
    )
    def k(s_hbm, r_hbm, proj_hbm, x_hbm, out_hbm, x_sp, *bufs):
        sid = lax.axis_index("s")
        wid = sid * nc + lax.axis_index("c")
        w0 = wid * per_w
        B = [bufs[i * 10:(i + 1) * 10] for i in range(_NBUF)]

        row0 = sid * rpt

        @pl.when(sid < ns - 1)
        def _():
            pltpu.sync_copy(x_hbm.at[pl.ds(row0, rpt)],
                            x_sp.at[pl.ds(row0, rpt)])

        @pl.when(sid == ns - 1)
        def _():
            pltpu.sync_copy(x_hbm.at[pl.ds((ns - 1) * rpt, rpt_last)],
                            x_sp.at[pl.ds((ns - 1) * rpt, rpt_last)])

        plsc.subcore_barrier()

        def stage(ci, b):
            sidx, ridx, rows_s, rows_r, projv, _, sem_gs, sem_gr, sem_proj, _ = B[b]
            base = w0 + ci * _K
            pltpu.sync_copy(s_hbm.at[pl.ds(base, _K)], sidx)
            pltpu.sync_copy(r_hbm.at[pl.ds(base, _K)], ridx)
            pltpu.async_copy(x_sp.at[sidx], rows_s, sem_gs)
            pltpu.async_copy(x_sp.at[ridx], rows_r, sem_gr)
            pltpu.async_copy(proj_hbm.at[pl.ds(base, _K)], projv, sem_proj)

        def wait_in(b):
            sidx, ridx, rows_s, rows_r, projv, _, sem_gs, sem_gr, sem_proj, _ = B[b]
            pltpu.make_async_copy(x_sp.at[sidx], rows_s, sem_gs).wait()
            pltpu.make_async_copy(x_sp.at[ridx], rows_r, sem_gr).wait()
            pltpu.make_async_copy(proj_hbm.at[pl.ds(0, _K)], projv, sem_proj).wait()

        def drain_out(b):
            outb = B[b][5]
            sem_out = B[b][9]
            pltpu.make_async_copy(outb, out_hbm.at[pl.ds(w0, _K)], sem_out).wait()

        def compute_and_out(ci, b):
            _, _, rows_s, rows_r, projv, outb, _, _, _, sem_out = B[b]

            def ebody(e, c2):
                for c8 in range(C // 16):
                    sl = pl.ds(c8 * 16, 16)
                    outb[e, sl] = (rows_s[e, sl] + rows_r[e, sl]) * projv[e, sl]
                return c2

            lax.fori_loop(0, _K, ebody, 0, unroll=2)
            base = w0 + ci * _K
            pltpu.async_copy(outb, out_hbm.at[pl.ds(base, _K)], sem_out)

        stage(0, 0)
        stage(1, 1)

        def outer(g, carry):
            for b in range(_NBUF):
                ci = g * _NBUF + b
                wait_in(b)

                @pl.when(ci >= _NBUF)
                def _():
                    drain_out(b)

                compute_and_out(ci, b)

                @pl.when(ci + _NBUF < n_chunk)
                def _():
                    stage(ci + _NBUF, b)
            return carry

        n_main = n_chunk // _NBUF
        lax.fori_loop(0, n_main, outer, 0)

        for ci_last in range(n_main * _NBUF, n_chunk):
            bl = ci_last % _NBUF
            wait_in(bl)
            drain_out(bl)
            compute_and_out(ci_last, bl)
        for b in range(_NBUF):
            drain_out(b)

    return k(senders, receivers, proj, x)


def kernel(senders, receivers, edge_attr, x, W, b):
    proj = _proj_tc(edge_attr, W, b)
    return _combine_sc(senders, receivers, proj, x)


# speedup vs baseline: 1.3704x; 1.3704x over previous
import functools

import jax
import jax.numpy as jnp
from jax import lax
from jax.experimental import pallas as pl
from jax.experimental.pallas import tpu as pltpu
from jax.experimental.pallas import tpu_sc as plsc

E = 320000
N = 10000
R = 16
C = 128


_BE = 8000


def _proj_body(ea_ref, w_ref, b_ref, o_ref):
    o_ref[...] = (
        jnp.dot(ea_ref[...], w_ref[...], preferred_element_type=jnp.float32)
        + b_ref[...]
    )


def _proj_tc(edge_attr, W, b):
    return pl.pallas_call(
        _proj_body,
        grid=(E // _BE,),
        in_specs=[
            pl.BlockSpec((_BE, R), lambda i: (i, 0)),
            pl.BlockSpec((R, C), lambda i: (0, 0)),
            pl.BlockSpec((1, C), lambda i: (0, 0)),
        ],
        out_specs=pl.BlockSpec((_BE, C), lambda i: (i, 0)),
        out_shape=jax.ShapeDtypeStruct((E, C), jnp.float32),
    )(edge_attr, W, b.reshape(1, C))


_K = 40
_NBUF = 2


def _combine_sc(senders, receivers, proj, x):
    info = plsc.get_sparse_core_info()
    nc = info.num_cores
    nw = nc * info.num_subcores
    per_w = E // nw
    n_chunk = per_w // _K

    mesh = plsc.VectorSubcoreMesh(core_axis_name="c", subcore_axis_name="s")

    ns = info.num_subcores
    rpt = (N // ns) // 8 * 8
    rpt_last = N - (ns - 1) * rpt

    scratch = [pltpu.VMEM_SHARED((N, C), jnp.float32)]
    for _ in range(_NBUF):
        scratch += [
            pltpu.VMEM((_K,), jnp.int32),
            pltpu.VMEM((_K,), jnp.int32),
            pltpu.VMEM((_K, C), jnp.float32),
            pltpu.VMEM((_K, C), jnp.float32),
            pltpu.VMEM((_K, C), jnp.float32),
            pltpu.VMEM((_K, C), jnp.float32),
            pltpu.SemaphoreType.DMA,
            pltpu.SemaphoreType.DMA,
            pltpu.SemaphoreType.DMA,
            pltpu.SemaphoreType.DMA,
        ]

    @functools.partial(
        pl.kernel,
        mesh=mesh,
        out_type=jax.ShapeDtypeStruct((E, C), jnp.float32),
        scratch_types=scratch,
    )
    def k(s_hbm, r_hbm, proj_hbm, x_hbm, out_hbm, x_sp, *bufs):
        sid = lax.axis_index("s")
        wid = sid * nc + lax.axis_index("c")
        w0 = wid * per_w
        B = [bufs[i * 10:(i + 1) * 10] for i in range(_NBUF)]

        row0 = sid * rpt

        @pl.when(sid < ns - 1)
        def _():
            pltpu.sync_copy(x_hbm.at[pl.ds(row0, rpt)],
                            x_sp.at[pl.ds(row0, rpt)])

        @pl.when(sid == ns - 1)
        def _():
            pltpu.sync_copy(x_hbm.at[pl.ds((ns - 1) * rpt, rpt_last)],
                            x_sp.at[pl.ds((ns - 1) * rpt, rpt_last)])

        plsc.subcore_barrier()

        def stage(ci, b):
            sidx, ridx, rows_s, rows_r, projv, _, sem_gs, sem_gr, sem_proj, _ = B[b]
            base = w0 + ci * _K
            pltpu.sync_copy(s_hbm.at[pl.ds(base, _K)], sidx)
            pltpu.sync_copy(r_hbm.at[pl.ds(base, _K)], ridx)
            pltpu.async_copy(x_sp.at[sidx], rows_s, sem_gs)
            pltpu.async_copy(x_sp.at[ridx], rows_r, sem_gr)
            pltpu.async_copy(proj_hbm.at[pl.ds(base, _K)], projv, sem_proj)

        def wait_in(b):
            sidx, ridx, rows_s, rows_r, projv, _, sem_gs, sem_gr, sem_proj, _ = B[b]
            pltpu.make_async_copy(x_sp.at[sidx], rows_s, sem_gs).wait()
            pltpu.make_async_copy(x_sp.at[ridx], rows_r, sem_gr).wait()
            pltpu.make_async_copy(proj_hbm.at[pl.ds(0, _K)], projv, sem_proj).wait()

        def drain_out(b):
            outb = B[b][5]
            sem_out = B[b][9]
            pltpu.make_async_copy(outb, out_hbm.at[pl.ds(w0, _K)], sem_out).wait()

        def compute_and_out(ci, b):
            _, _, rows_s, rows_r, projv, outb, _, _, _, sem_out = B[b]

            def ebody(e, c2):
                for c8 in range(C // 16):
                    sl = pl.ds(c8 * 16, 16)
                    outb[e, sl] = (rows_s[e, sl] + rows_r[e, sl]) * projv[e, sl]
                return c2

            lax.fori_loop(0, _K, ebody, 0, unroll=8)
            base = w0 + ci * _K
            pltpu.async_copy(outb, out_hbm.at[pl.ds(base, _K)], sem_out)

        stage(0, 0)
        stage(1, 1)

        def outer(g, carry):
            for b in range(_NBUF):
                ci = g * _NBUF + b
                wait_in(b)

                @pl.when(ci >= _NBUF)
                def _():
                    drain_out(b)

                compute_and_out(ci, b)

                @pl.when(ci + _NBUF < n_chunk)
                def _():
                    stage(ci + _NBUF, b)
            return carry

        n_main = n_chunk // _NBUF
        lax.fori_loop(0, n_main, outer, 0)

        for ci_last in range(n_main * _NBUF, n_chunk):
            bl = ci_last % _NBUF
            wait_in(bl)
            drain_out(bl)
            compute_and_out(ci_last, bl)
        for b in range(_NBUF):
            drain_out(b)

    return k(senders, receivers, proj, x)


def kernel(senders, receivers, edge_attr, x, W, b):
    proj = _proj_tc(edge_attr, W, b)
    return _combine_sc(senders, receivers, proj, x)


# speedup vs baseline: 1.9589x; 1.4294x over previous
import functools

import jax
import jax.numpy as jnp
from jax import lax
from jax.experimental import pallas as pl
from jax.experimental.pallas import tpu as pltpu
from jax.experimental.pallas import tpu_sc as plsc

E = 320000
N = 10000
R = 16
C = 128


_BE = 8000


def _proj_body(ea_ref, w_ref, b_ref, o_ref):
    o_ref[...] = (
        jnp.dot(ea_ref[...], w_ref[...], preferred_element_type=jnp.float32)
        + b_ref[...]
    )


def _proj_tc(edge_attr, W, b):
    return pl.pallas_call(
        _proj_body,
        grid=(E // _BE,),
        in_specs=[
            pl.BlockSpec((_BE, R), lambda i: (i, 0)),
            pl.BlockSpec((R, C), lambda i: (0, 0)),
            pl.BlockSpec((1, C), lambda i: (0, 0)),
        ],
        out_specs=pl.BlockSpec((_BE, C), lambda i: (i, 0)),
        out_shape=jax.ShapeDtypeStruct((E, C), jnp.float32),
    )(edge_attr, W, b.reshape(1, C))


_K = 40
_NBUF = 2


def _combine_sc(senders, receivers, proj, x):
    info = plsc.get_sparse_core_info()
    nc = info.num_cores
    nw = nc * info.num_subcores
    per_w = E // nw
    n_chunk = per_w // _K

    mesh = plsc.VectorSubcoreMesh(core_axis_name="c", subcore_axis_name="s")

    ns = info.num_subcores
    rpt = (N // ns) // 8 * 8
    rpt_last = N - (ns - 1) * rpt

    scratch = [pltpu.VMEM_SHARED((N, C), jnp.float32)]
    for _ in range(_NBUF):
        scratch += [
            pltpu.VMEM((_K, C), jnp.float32),
            pltpu.VMEM((_K, C), jnp.float32),
            pltpu.VMEM((_K, C), jnp.float32),
            pltpu.VMEM((_K, C), jnp.float32),
            pltpu.SemaphoreType.DMA,
            pltpu.SemaphoreType.DMA,
            pltpu.SemaphoreType.DMA,
            pltpu.SemaphoreType.DMA,
        ]
    _NIDX = 2 * _NBUF
    for _ in range(_NIDX):
        scratch += [
            pltpu.VMEM((_K,), jnp.int32),
            pltpu.VMEM((_K,), jnp.int32),
            pltpu.SemaphoreType.DMA,
            pltpu.SemaphoreType.DMA,
        ]

    @functools.partial(
        pl.kernel,
        mesh=mesh,
        out_type=jax.ShapeDtypeStruct((E, C), jnp.float32),
        scratch_types=scratch,
    )
    def k(s_hbm, r_hbm, proj_hbm, x_hbm, out_hbm, x_sp, *bufs):
        sid = lax.axis_index("s")
        wid = sid * nc + lax.axis_index("c")
        w0 = wid * per_w
        nidx = 2 * _NBUF
        B = [bufs[i * 8:(i + 1) * 8] for i in range(_NBUF)]
        ib = bufs[_NBUF * 8:]
        I = [ib[i * 4:(i + 1) * 4] for i in range(nidx)]

        row0 = sid * rpt

        @pl.when(sid < ns - 1)
        def _():
            pltpu.sync_copy(x_hbm.at[pl.ds(row0, rpt)],
                            x_sp.at[pl.ds(row0, rpt)])

        @pl.when(sid == ns - 1)
        def _():
            pltpu.sync_copy(x_hbm.at[pl.ds((ns - 1) * rpt, rpt_last)],
                            x_sp.at[pl.ds((ns - 1) * rpt, rpt_last)])

        plsc.subcore_barrier()

        def stage_idx(ci, j):
            sidx, ridx, sem_is, sem_ir = I[j]
            base = w0 + ci * _K
            pltpu.async_copy(s_hbm.at[pl.ds(base, _K)], sidx, sem_is)
            pltpu.async_copy(r_hbm.at[pl.ds(base, _K)], ridx, sem_ir)

        def wait_idx(j):
            sidx, ridx, sem_is, sem_ir = I[j]
            pltpu.make_async_copy(s_hbm.at[pl.ds(0, _K)], sidx, sem_is).wait()
            pltpu.make_async_copy(r_hbm.at[pl.ds(0, _K)], ridx, sem_ir).wait()

        def stage_gather(ci, b, j):
            rows_s, rows_r, projv, _, sem_gs, sem_gr, sem_proj, _ = B[b]
            sidx, ridx, _, _ = I[j]
            base = w0 + ci * _K
            pltpu.async_copy(x_sp.at[sidx], rows_s, sem_gs)
            pltpu.async_copy(x_sp.at[ridx], rows_r, sem_gr)
            pltpu.async_copy(proj_hbm.at[pl.ds(base, _K)], projv, sem_proj)

        def wait_in(b, j):
            rows_s, rows_r, projv, _, sem_gs, sem_gr, sem_proj, _ = B[b]
            sidx, ridx, _, _ = I[j]
            pltpu.make_async_copy(x_sp.at[sidx], rows_s, sem_gs).wait()
            pltpu.make_async_copy(x_sp.at[ridx], rows_r, sem_gr).wait()
            pltpu.make_async_copy(proj_hbm.at[pl.ds(0, _K)], projv, sem_proj).wait()

        def drain_out(b):
            outb = B[b][3]
            sem_out = B[b][7]
            pltpu.make_async_copy(outb, out_hbm.at[pl.ds(w0, _K)], sem_out).wait()

        def compute_and_out(ci, b):
            rows_s, rows_r, projv, outb, _, _, _, sem_out = B[b]

            def ebody(e, c2):
                for c8 in range(C // 16):
                    sl = pl.ds(c8 * 16, 16)
                    outb[e, sl] = (rows_s[e, sl] + rows_r[e, sl]) * projv[e, sl]
                return c2

            lax.fori_loop(0, _K, ebody, 0, unroll=8)
            base = w0 + ci * _K
            pltpu.async_copy(outb, out_hbm.at[pl.ds(base, _K)], sem_out)

        for ci in range(nidx):
            stage_idx(ci, ci % nidx)
        for ci in range(_NBUF):
            wait_idx(ci % nidx)
            stage_gather(ci, ci % _NBUF, ci % nidx)

        def outer(g, carry):
            for u in range(nidx):
                ci = g * nidx + u
                b = u % _NBUF
                wait_in(b, u)

                @pl.when(ci >= _NBUF)
                def _():
                    drain_out(b)

                compute_and_out(ci, b)

                @pl.when(ci + _NBUF < n_chunk)
                def _():
                    wait_idx((u + _NBUF) % nidx)
                    stage_gather(ci + _NBUF, b, (u + _NBUF) % nidx)

                @pl.when(ci + nidx < n_chunk)
                def _():
                    stage_idx(ci + nidx, u)
            return carry

        n_main = n_chunk // nidx
        lax.fori_loop(0, n_main, outer, 0)

        for ci_last in range(n_main * nidx, n_chunk):
            u = ci_last % nidx
            bl = ci_last % _NBUF
            wait_in(bl, u)
            drain_out(bl)
            compute_and_out(ci_last, bl)
        for b in range(_NBUF):
            drain_out(b)

    return k(senders, receivers, proj, x)


def kernel(senders, receivers, edge_attr, x, W, b):
    proj = _proj_tc(edge_attr, W, b)
    return _combine_sc(senders, receivers, proj, x)


# speedup vs baseline: 1.9946x; 1.0182x over previous
import functools

import jax
import jax.numpy as jnp
from jax import lax
from jax.experimental import pallas as pl
from jax.experimental.pallas import tpu as pltpu
from jax.experimental.pallas import tpu_sc as plsc

E = 320000
N = 10000
R = 16
C = 128

_NSPLIT = 2
_EH = E // _NSPLIT


_BE = 8000


def _proj_body(ea_ref, w_ref, b_ref, o_ref):
    o_ref[...] = (
        jnp.dot(ea_ref[...], w_ref[...], preferred_element_type=jnp.float32)
        + b_ref[...]
    )


def _proj_tc(edge_attr, W, b, e0):
    blk0 = e0 // _BE
    return pl.pallas_call(
        _proj_body,
        grid=(_EH // _BE,),
        in_specs=[
            pl.BlockSpec((_BE, R), lambda i: (i + blk0, 0)),
            pl.BlockSpec((R, C), lambda i: (0, 0)),
            pl.BlockSpec((1, C), lambda i: (0, 0)),
        ],
        out_specs=pl.BlockSpec((_BE, C), lambda i: (i, 0)),
        out_shape=jax.ShapeDtypeStruct((_EH, C), jnp.float32),
    )(edge_attr, W, b.reshape(1, C))


_K = 40
_NBUF = 2
_NIDX = 2 * _NBUF


def _sc_scratch():
    scratch = [pltpu.VMEM_SHARED((N, C), jnp.float32)]
    for _ in range(_NBUF):
        scratch += [
            pltpu.VMEM((_K, C), jnp.float32),
            pltpu.VMEM((_K, C), jnp.float32),
            pltpu.VMEM((_K, C), jnp.float32),
            pltpu.VMEM((_K, C), jnp.float32),
            pltpu.SemaphoreType.DMA,
            pltpu.SemaphoreType.DMA,
            pltpu.SemaphoreType.DMA,
            pltpu.SemaphoreType.DMA,
        ]
    for _ in range(_NIDX):
        scratch += [
            pltpu.VMEM((_K,), jnp.int32),
            pltpu.VMEM((_K,), jnp.int32),
            pltpu.SemaphoreType.DMA,
            pltpu.SemaphoreType.DMA,
        ]
    return scratch


def _sc_body(e0, s_hbm, r_hbm, proj_hbm, x_hbm, out_hbm, x_sp, bufs):
    info = plsc.get_sparse_core_info()
    nc = info.num_cores
    ns = info.num_subcores
    nw = nc * ns
    per_w = _EH // nw
    n_chunk = per_w // _K

    sid = lax.axis_index("s")
    wid = sid * nc + lax.axis_index("c")
    w0l = wid * per_w
    w0g = e0 + w0l
    B = [bufs[i * 8:(i + 1) * 8] for i in range(_NBUF)]
    ib = bufs[_NBUF * 8:]
    I = [ib[i * 4:(i + 1) * 4] for i in range(_NIDX)]

    rpt = (N // ns) // 8 * 8
    rpt_last = N - (ns - 1) * rpt
    row0 = sid * rpt

    @pl.when(sid < ns - 1)
    def _():
        pltpu.sync_copy(x_hbm.at[pl.ds(row0, rpt)], x_sp.at[pl.ds(row0, rpt)])

    @pl.when(sid == ns - 1)
    def _():
        pltpu.sync_copy(x_hbm.at[pl.ds((ns - 1) * rpt, rpt_last)],
                        x_sp.at[pl.ds((ns - 1) * rpt, rpt_last)])

    plsc.subcore_barrier()

    def stage_idx(ci, j):
        sidx, ridx, sem_is, sem_ir = I[j]
        base = w0g + ci * _K
        pltpu.async_copy(s_hbm.at[pl.ds(base, _K)], sidx, sem_is)
        pltpu.async_copy(r_hbm.at[pl.ds(base, _K)], ridx, sem_ir)

    def wait_idx(j):
        sidx, ridx, sem_is, sem_ir = I[j]
        pltpu.make_async_copy(s_hbm.at[pl.ds(0, _K)], sidx, sem_is).wait()
        pltpu.make_async_copy(r_hbm.at[pl.ds(0, _K)], ridx, sem_ir).wait()

    def stage_gather(ci, b, j):
        rows_s, rows_r, projv, _, sem_gs, sem_gr, sem_proj, _ = B[b]
        sidx, ridx, _, _ = I[j]
        pltpu.async_copy(x_sp.at[sidx], rows_s, sem_gs)
        pltpu.async_copy(x_sp.at[ridx], rows_r, sem_gr)
        pltpu.async_copy(proj_hbm.at[pl.ds(w0l + ci * _K, _K)], projv, sem_proj)

    def wait_in(b, j):
        rows_s, rows_r, projv, _, sem_gs, sem_gr, sem_proj, _ = B[b]
        sidx, ridx, _, _ = I[j]
        pltpu.make_async_copy(x_sp.at[sidx], rows_s, sem_gs).wait()
        pltpu.make_async_copy(x_sp.at[ridx], rows_r, sem_gr).wait()
        pltpu.make_async_copy(proj_hbm.at[pl.ds(0, _K)], projv, sem_proj).wait()

    def drain_out(b):
        outb = B[b][3]
        sem_out = B[b][7]
        pltpu.make_async_copy(outb, out_hbm.at[pl.ds(w0g, _K)], sem_out).wait()

    def compute_and_out(ci, b):
        rows_s, rows_r, projv, outb, _, _, _, sem_out = B[b]

        def ebody(e, c2):
            for c8 in range(C // 16):
                sl = pl.ds(c8 * 16, 16)
                outb[e, sl] = (rows_s[e, sl] + rows_r[e, sl]) * projv[e, sl]
            return c2

        lax.fori_loop(0, _K, ebody, 0, unroll=8)
        pltpu.async_copy(outb, out_hbm.at[pl.ds(w0g + ci * _K, _K)], sem_out)

    for ci in range(_NIDX):
        stage_idx(ci, ci % _NIDX)
    for ci in range(_NBUF):
        wait_idx(ci % _NIDX)
        stage_gather(ci, ci % _NBUF, ci % _NIDX)

    def outer(g, carry):
        for u in range(_NIDX):
            ci = g * _NIDX + u
            b = u % _NBUF
            wait_in(b, u)

            @pl.when(ci >= _NBUF)
            def _():
                drain_out(b)

            compute_and_out(ci, b)

            @pl.when(ci + _NBUF < n_chunk)
            def _():
                wait_idx((u + _NBUF) % _NIDX)
                stage_gather(ci + _NBUF, b, (u + _NBUF) % _NIDX)

            @pl.when(ci + _NIDX < n_chunk)
            def _():
                stage_idx(ci + _NIDX, u)
        return carry

    n_main = n_chunk // _NIDX
    lax.fori_loop(0, n_main, outer, 0)

    for ci_last in range(n_main * _NIDX, n_chunk):
        u = ci_last % _NIDX
        bl = ci_last % _NBUF
        wait_in(bl, u)
        drain_out(bl)
        compute_and_out(ci_last, bl)
    for b in range(_NBUF):
        drain_out(b)


def _combine_first(senders, receivers, proj, x):
    mesh = plsc.VectorSubcoreMesh(core_axis_name="c", subcore_axis_name="s")

    @functools.partial(
        pl.kernel,
        mesh=mesh,
        out_type=jax.ShapeDtypeStruct((E, C), jnp.float32),
        scratch_types=_sc_scratch(),
    )
    def k(s_hbm, r_hbm, proj_hbm, x_hbm, out_hbm, x_sp, *bufs):
        _sc_body(0, s_hbm, r_hbm, proj_hbm, x_hbm, out_hbm, x_sp, bufs)

    return k(senders, receivers, proj, x)


def _combine_into(e0, senders, receivers, proj, x, out_ref):
    mesh = plsc.VectorSubcoreMesh(core_axis_name="c", subcore_axis_name="s")

    @functools.partial(
        pl.kernel,
        mesh=mesh,
        out_type=(),
        scratch_types=_sc_scratch(),
    )
    def k(s_hbm, r_hbm, proj_hbm, x_hbm, out_hbm, x_sp, *bufs):
        _sc_body(e0, s_hbm, r_hbm, proj_hbm, x_hbm, out_hbm, x_sp, bufs)

    k(senders, receivers, proj, x, out_ref)


def kernel(senders, receivers, edge_attr, x, W, b):
    projs = [_proj_tc(edge_attr, W, b, i * _EH) for i in range(_NSPLIT)]
    out0 = _combine_first(senders, receivers, projs[0], x)
    ref = jax.new_ref(out0)
    for i in range(1, _NSPLIT):
        _combine_into(i * _EH, senders, receivers, projs[i], x, ref)
    return ref[...]
